# baseline (speedup 1.0000x reference)
import functools

import jax
import jax.numpy as jnp
from jax import lax
from jax.experimental import pallas as pl
from jax.experimental.pallas import tpu as pltpu
from jax.experimental.pallas import tpu_sc as plsc

B, INPUT_DIM, K, L, D_C = 16384, 256, 4, 1024, 64
D_PAD = 128
BM = 512


def _encode(x_ref, w1_ref, b1_ref, w2_ref, b2_ref, w3_ref, b3_ref,
            g_ref, bb_ref, cb_ref,
            toks_ref, gidx_ref, e_ref, cbt_ref, cbn_ref):
    pid = pl.program_id(0)
    x = x_ref[...]
    g = g_ref[...]
    bb = bb_ref[...]

    @pl.when(pid == 0)
    def _norm_codebooks():
        for k in range(K):
            cb = cb_ref[k]
            cbn = cb * (1.0 / jnp.maximum(
                jnp.sqrt(jnp.sum(cb * cb, axis=-1, keepdims=True)), 1e-12))
            cbn_ref[k] = cbn
            cbt_ref[k * L:(k + 1) * L, :] = jnp.concatenate(
                [cbn, jnp.zeros((L, D_PAD - D_C), jnp.float32)], axis=1)

    idx_cols = []
    e_cols = []
    for k in range(K):
        h = jnp.maximum(x @ w1_ref[k] + b1_ref[k], 0.0)
        h = jnp.maximum(h @ w2_ref[k] + b2_ref[k], 0.0)
        e = h @ w3_ref[k] + b3_ref[k]
        m = e.mean(axis=-1, keepdims=True)
        v = ((e - m) ** 2).mean(axis=-1, keepdims=True)
        e = (e - m) * (1.0 / jnp.sqrt(v + 1e-5)) * g + bb
        e_cols.append(e)
        en = e * (1.0 / jnp.maximum(
            jnp.sqrt(jnp.sum(e * e, axis=-1, keepdims=True)), 1e-12))
        sim = jax.lax.dot_general(en, cbn_ref[k], (((1,), (1,)), ((), ())))
        dist = 1.0 - sim
        idx = jnp.argmin(dist, axis=1).reshape(BM, 1).astype(jnp.int32)
        idx_cols.append(idx)

    toks = jnp.concatenate(idx_cols, axis=1)
    toks_ref[...] = toks
    off = jax.lax.broadcasted_iota(jnp.int32, (BM, K), 1) * L
    gidx_ref[...] = toks + off
    e_ref[...] = jnp.concatenate(e_cols, axis=1)


def _encode_call(x, enc_w1, b1, enc_w2, b2, enc_w3, b3, g, bb, codebooks):
    def rep(a):
        return pl.BlockSpec(a.shape, lambda i: (0,) * a.ndim)

    return pl.pallas_call(
        _encode,
        grid=(B // BM,),
        in_specs=[
            pl.BlockSpec((BM, INPUT_DIM), lambda i: (i, 0)),
            rep(enc_w1), rep(b1), rep(enc_w2), rep(b2), rep(enc_w3), rep(b3),
            rep(g), rep(bb), rep(codebooks),
        ],
        out_specs=[
            pl.BlockSpec((BM, K), lambda i: (i, 0)),
            pl.BlockSpec((BM, K), lambda i: (i, 0)),
            pl.BlockSpec((BM, K * D_C), lambda i: (i, 0)),
            pl.BlockSpec((K * L, D_PAD), lambda i: (0, 0)),
        ],
        out_shape=[
            jax.ShapeDtypeStruct((B, K), jnp.int32),
            jax.ShapeDtypeStruct((B, K), jnp.int32),
            jax.ShapeDtypeStruct((B, K * D_C), jnp.float32),
            jax.ShapeDtypeStruct((K * L, D_PAD), jnp.float32),
        ],
        scratch_shapes=[pltpu.VMEM((K, L, D_C), jnp.float32)],
    )(x, enc_w1, b1, enc_w2, b2, enc_w3, b3, g, bb, codebooks)


_NW = 32
_ROWS_W = (B * K) // _NW
_CH = 512


@functools.partial(
    pl.kernel,
    mesh=plsc.VectorSubcoreMesh(core_axis_name="c", subcore_axis_name="s"),
    out_type=jax.ShapeDtypeStruct((B * K, D_PAD), jnp.float32),
    scratch_types=[
        pltpu.VMEM((_CH,), jnp.int32),
        pltpu.VMEM((_CH, D_PAD), jnp.float32),
        pltpu.SemaphoreType.DMA,
    ],
)
def _sc_gather(cbt_hbm, gidx_hbm, out_hbm, idx_v, rows_v, sem):
    wid = lax.axis_index("s") * 2 + lax.axis_index("c")
    for c in range(_ROWS_W // _CH):
        base = wid * _ROWS_W + c * _CH
        pltpu.sync_copy(gidx_hbm.at[pl.ds(base, _CH)], idx_v)
        pltpu.async_copy(cbt_hbm.at[idx_v], rows_v, sem).wait()
        pltpu.sync_copy(rows_v, out_hbm.at[pl.ds(base, _CH)])


def _decode(x_ref, e_ref, q_ref, dw1_ref, db1_ref, dw2_ref, db2_ref,
            dw3_ref, db3_ref, rec_ref, rl_ref, cl_ref):
    pid = pl.program_id(0)
    x = x_ref[...]
    q_all = q_ref[...].reshape(BM, K, D_PAD)

    acc_q = jnp.zeros((BM, D_C), jnp.float32)
    cb_loss = jnp.float32(0.0)
    for k in range(K):
        e = e_ref[:, k * D_C:(k + 1) * D_C]
        q = q_all[:, k, :D_C]
        q_st = e + (q - e)
        acc_q = acc_q + q_st
        d = e - q_st
        cb_loss = cb_loss + jnp.sum(d * d)

    avg_q = acc_q * (1.0 / K)
    h = jnp.maximum(avg_q @ dw1_ref[...] + db1_ref[...], 0.0)
    h = jnp.maximum(h @ dw2_ref[...] + db2_ref[...], 0.0)
    rec = h @ dw3_ref[...] + db3_ref[...]
    rec_ref[...] = rec

    r = rec - x
    rl_blk = jnp.sum(r * r) * (1.0 / (B * INPUT_DIM))
    cl_blk = cb_loss * (1.0 / (B * D_C))

    @pl.when(pid == 0)
    def _init():
        rl_ref[...] = jnp.zeros((1, 1), jnp.float32)
        cl_ref[...] = jnp.zeros((1, 1), jnp.float32)

    rl_ref[...] += jnp.reshape(rl_blk, (1, 1))
    cl_ref[...] += jnp.reshape(cl_blk, (1, 1))


def _decode_call(x, e, q, dec_w1, db1, dec_w2, db2, dec_w3, db3):
    def rep(a):
        return pl.BlockSpec(a.shape, lambda i: (0,) * a.ndim)

    return pl.pallas_call(
        _decode,
        grid=(B // BM,),
        in_specs=[
            pl.BlockSpec((BM, INPUT_DIM), lambda i: (i, 0)),
            pl.BlockSpec((BM, K * D_C), lambda i: (i, 0)),
            pl.BlockSpec((BM * K, D_PAD), lambda i: (i, 0)),
            rep(dec_w1), rep(db1), rep(dec_w2), rep(db2), rep(dec_w3), rep(db3),
        ],
        out_specs=[
            pl.BlockSpec((BM, INPUT_DIM), lambda i: (i, 0)),
            pl.BlockSpec((1, 1), lambda i: (0, 0)),
            pl.BlockSpec((1, 1), lambda i: (0, 0)),
        ],
        out_shape=[
            jax.ShapeDtypeStruct((B, INPUT_DIM), jnp.float32),
            jax.ShapeDtypeStruct((1, 1), jnp.float32),
            jax.ShapeDtypeStruct((1, 1), jnp.float32),
        ],
    )(x, e, q, dec_w1, db1, dec_w2, db2, dec_w3, db3)


def kernel(x, enc_w1, enc_b1, enc_w2, enc_b2, enc_w3, enc_b3, ln_g, ln_b,
           codebooks, dec_w1, dec_b1, dec_w2, dec_b2, dec_w3, dec_b3):
    b1 = enc_b1[:, None, :]
    b2 = enc_b2[:, None, :]
    b3 = enc_b3[:, None, :]
    g = ln_g[None, :]
    bb = ln_b[None, :]
    db1 = dec_b1[None, :]
    db2 = dec_b2[None, :]
    db3 = dec_b3[None, :]

    toks, gidx, e, cbt = _encode_call(
        x, enc_w1, b1, enc_w2, b2, enc_w3, b3, g, bb, codebooks)
    q = _sc_gather(cbt, gidx.reshape(B * K))
    rec, rl, cl = _decode_call(x, e, q, dec_w1, db1, dec_w2, db2, dec_w3, db3)

    rl_s = rl[0, 0]
    cl_s = cl[0, 0]
    return toks, rec, rl_s, cl_s, cl_s


# speedup vs baseline: 2.7757x; 2.7757x over previous
import jax
import jax.numpy as jnp
from jax.experimental import pallas as pl
from jax.experimental.pallas import tpu as pltpu

B, INPUT_DIM, K, L, D_C = 16384, 256, 4, 1024, 64
BM = 1024


def _fused(x_ref, w1_ref, b1_ref, w2_ref, b2_ref, w3_ref, b3_ref,
           g_ref, bb_ref, cb_ref, dw1_ref, db1_ref, dw2_ref, db2_ref,
           dw3_ref, db3_ref,
           toks_ref, rec_ref, rl_ref, cl_ref, cbn_ref):
    pid = pl.program_id(0)
    x = x_ref[...]
    g = g_ref[...]
    bb = bb_ref[...]

    @pl.when(pid == 0)
    def _norm_codebooks():
        for k in range(K):
            cb = cb_ref[k]
            cbn_ref[k] = cb / jnp.maximum(
                jnp.sqrt(jnp.sum(cb * cb, axis=-1, keepdims=True)), 1e-12)

    acc_q = jnp.zeros((BM, D_C), jnp.float32)
    cb_loss = jnp.float32(0.0)
    idx_cols = []
    for k in range(K):
        h = jnp.maximum(x @ w1_ref[k] + b1_ref[k], 0.0)
        h = jnp.maximum(h @ w2_ref[k] + b2_ref[k], 0.0)
        e = h @ w3_ref[k] + b3_ref[k]
        m = e.mean(axis=-1, keepdims=True)
        v = ((e - m) ** 2).mean(axis=-1, keepdims=True)
        e = (e - m) * (1.0 / jnp.sqrt(v + 1e-5)) * g + bb
        en = e * (1.0 / jnp.maximum(
            jnp.sqrt(jnp.sum(e * e, axis=-1, keepdims=True)), 1e-12))
        cbn = cbn_ref[k]
        sim = jax.lax.dot_general(en, cbn, (((1,), (1,)), ((), ())))
        dist = 1.0 - sim
        idx = jnp.argmin(dist, axis=1).reshape(BM, 1).astype(jnp.int32)
        iota = jax.lax.broadcasted_iota(jnp.int32, (BM, L), 1)
        idx_cols.append(idx)
        onehot = (iota == idx).astype(jnp.float32)
        q = jax.lax.dot_general(onehot, cbn, (((1,), (0,)), ((), ())))
        q_st = e + (q - e)
        acc_q = acc_q + q_st
        d = e - q_st
        cb_loss = cb_loss + jnp.sum(d * d)

    avg_q = acc_q * (1.0 / K)
    h = jnp.maximum(avg_q @ dw1_ref[...] + db1_ref[...], 0.0)
    h = jnp.maximum(h @ dw2_ref[...] + db2_ref[...], 0.0)
    rec = h @ dw3_ref[...] + db3_ref[...]
    rec_ref[...] = rec
    toks_ref[...] = jnp.concatenate(idx_cols, axis=1)

    r = rec - x
    rl_blk = jnp.sum(r * r) * (1.0 / (B * INPUT_DIM))
    cl_blk = cb_loss * (1.0 / (B * D_C))

    @pl.when(pid == 0)
    def _init():
        rl_ref[...] = jnp.zeros((1, 1), jnp.float32)
        cl_ref[...] = jnp.zeros((1, 1), jnp.float32)

    rl_ref[...] += jnp.reshape(rl_blk, (1, 1))
    cl_ref[...] += jnp.reshape(cl_blk, (1, 1))


def kernel(x, enc_w1, enc_b1, enc_w2, enc_b2, enc_w3, enc_b3, ln_g, ln_b,
           codebooks, dec_w1, dec_b1, dec_w2, dec_b2, dec_w3, dec_b3):
    b1 = enc_b1[:, None, :]
    b2 = enc_b2[:, None, :]
    b3 = enc_b3[:, None, :]
    g = ln_g[None, :]
    bb = ln_b[None, :]
    db1 = dec_b1[None, :]
    db2 = dec_b2[None, :]
    db3 = dec_b3[None, :]

    def rep(a):
        return pl.BlockSpec(a.shape, lambda i: (0,) * a.ndim)

    grid = B // BM
    toks, rec, rl, cl = pl.pallas_call(
        _fused,
        grid=(grid,),
        in_specs=[
            pl.BlockSpec((BM, INPUT_DIM), lambda i: (i, 0)),
            rep(enc_w1), rep(b1), rep(enc_w2), rep(b2), rep(enc_w3), rep(b3),
            rep(g), rep(bb), rep(codebooks),
            rep(dec_w1), rep(db1), rep(dec_w2), rep(db2), rep(dec_w3), rep(db3),
        ],
        out_specs=[
            pl.BlockSpec((BM, K), lambda i: (i, 0)),
            pl.BlockSpec((BM, INPUT_DIM), lambda i: (i, 0)),
            pl.BlockSpec((1, 1), lambda i: (0, 0)),
            pl.BlockSpec((1, 1), lambda i: (0, 0)),
        ],
        out_shape=[
            jax.ShapeDtypeStruct((B, K), jnp.int32),
            jax.ShapeDtypeStruct((B, INPUT_DIM), jnp.float32),
            jax.ShapeDtypeStruct((1, 1), jnp.float32),
            jax.ShapeDtypeStruct((1, 1), jnp.float32),
        ],
        scratch_shapes=[pltpu.VMEM((K, L, D_C), jnp.float32)],
    )(x, enc_w1, b1, enc_w2, b2, enc_w3, b3, g, bb, codebooks,
      dec_w1, db1, dec_w2, db2, dec_w3, db3)

    rl_s = rl[0, 0]
    cl_s = cl[0, 0]
    return toks, rec, rl_s, cl_s, cl_s


# speedup vs baseline: 2.9491x; 1.0625x over previous
import jax
import jax.numpy as jnp
from jax.experimental import pallas as pl
from jax.experimental.pallas import tpu as pltpu

B, INPUT_DIM, K, L, D_C = 16384, 256, 4, 1024, 64
BM = 2048


def _fused(x_ref, w1_ref, b1_ref, w2_ref, b2_ref, w3_ref, b3_ref,
           g_ref, bb_ref, cb_ref, dw1_ref, db1_ref, dw2_ref, db2_ref,
           dw3_ref, db3_ref,
           toks_ref, rec_ref, rl_ref, cl_ref, cbn_ref):
    pid = pl.program_id(0)
    x = x_ref[...]
    g = g_ref[...]
    bb = bb_ref[...]

    @pl.when(pid == 0)
    def _norm_codebooks():
        for k in range(K):
            cb = cb_ref[k]
            cbn_ref[k] = cb / jnp.maximum(
                jnp.sqrt(jnp.sum(cb * cb, axis=-1, keepdims=True)), 1e-12)

    acc_q = jnp.zeros((BM, D_C), jnp.float32)
    cb_loss = jnp.float32(0.0)
    idx_cols = []
    for k in range(K):
        h = jnp.maximum(x @ w1_ref[k] + b1_ref[k], 0.0)
        h = jnp.maximum(h @ w2_ref[k] + b2_ref[k], 0.0)
        e = h @ w3_ref[k] + b3_ref[k]
        m = e.mean(axis=-1, keepdims=True)
        v = ((e - m) ** 2).mean(axis=-1, keepdims=True)
        e = (e - m) * (1.0 / jnp.sqrt(v + 1e-5)) * g + bb
        en = e * (1.0 / jnp.maximum(
            jnp.sqrt(jnp.sum(e * e, axis=-1, keepdims=True)), 1e-12))
        cbn = cbn_ref[k]
        sim = jax.lax.dot_general(en, cbn, (((1,), (1,)), ((), ())))
        dist = 1.0 - sim
        idx = jnp.argmin(dist, axis=1).reshape(BM, 1).astype(jnp.int32)
        iota = jax.lax.broadcasted_iota(jnp.int32, (BM, L), 1)
        idx_cols.append(idx)
        onehot = (iota == idx).astype(jnp.float32)
        q = jax.lax.dot_general(onehot, cbn, (((1,), (0,)), ((), ())))
        q_st = e + (q - e)
        acc_q = acc_q + q_st
        d = e - q_st
        cb_loss = cb_loss + jnp.sum(d * d)

    avg_q = acc_q * (1.0 / K)
    h = jnp.maximum(avg_q @ dw1_ref[...] + db1_ref[...], 0.0)
    h = jnp.maximum(h @ dw2_ref[...] + db2_ref[...], 0.0)
    rec = h @ dw3_ref[...] + db3_ref[...]
    rec_ref[...] = rec
    toks_ref[...] = jnp.concatenate(idx_cols, axis=1)

    r = rec - x
    rl_blk = jnp.sum(r * r) * (1.0 / (B * INPUT_DIM))
    cl_blk = cb_loss * (1.0 / (B * D_C))

    @pl.when(pid == 0)
    def _init():
        rl_ref[...] = jnp.zeros((1, 1), jnp.float32)
        cl_ref[...] = jnp.zeros((1, 1), jnp.float32)

    rl_ref[...] += jnp.reshape(rl_blk, (1, 1))
    cl_ref[...] += jnp.reshape(cl_blk, (1, 1))


def kernel(x, enc_w1, enc_b1, enc_w2, enc_b2, enc_w3, enc_b3, ln_g, ln_b,
           codebooks, dec_w1, dec_b1, dec_w2, dec_b2, dec_w3, dec_b3):
    b1 = enc_b1[:, None, :]
    b2 = enc_b2[:, None, :]
    b3 = enc_b3[:, None, :]
    g = ln_g[None, :]
    bb = ln_b[None, :]
    db1 = dec_b1[None, :]
    db2 = dec_b2[None, :]
    db3 = dec_b3[None, :]

    def rep(a):
        return pl.BlockSpec(a.shape, lambda i: (0,) * a.ndim)

    grid = B // BM
    toks, rec, rl, cl = pl.pallas_call(
        _fused,
        grid=(grid,),
        in_specs=[
            pl.BlockSpec((BM, INPUT_DIM), lambda i: (i, 0)),
            rep(enc_w1), rep(b1), rep(enc_w2), rep(b2), rep(enc_w3), rep(b3),
            rep(g), rep(bb), rep(codebooks),
            rep(dec_w1), rep(db1), rep(dec_w2), rep(db2), rep(dec_w3), rep(db3),
        ],
        out_specs=[
            pl.BlockSpec((BM, K), lambda i: (i, 0)),
            pl.BlockSpec((BM, INPUT_DIM), lambda i: (i, 0)),
            pl.BlockSpec((1, 1), lambda i: (0, 0)),
            pl.BlockSpec((1, 1), lambda i: (0, 0)),
        ],
        out_shape=[
            jax.ShapeDtypeStruct((B, K), jnp.int32),
            jax.ShapeDtypeStruct((B, INPUT_DIM), jnp.float32),
            jax.ShapeDtypeStruct((1, 1), jnp.float32),
            jax.ShapeDtypeStruct((1, 1), jnp.float32),
        ],
        scratch_shapes=[pltpu.VMEM((K, L, D_C), jnp.float32)],
    )(x, enc_w1, b1, enc_w2, b2, enc_w3, b3, g, bb, codebooks,
      dec_w1, db1, dec_w2, db2, dec_w3, db3)

    rl_s = rl[0, 0]
    cl_s = cl[0, 0]
    return toks, rec, rl_s, cl_s, cl_s
